# Initial kernel scaffold; baseline (speedup 1.0000x reference)
import functools

import jax
import jax.numpy as jnp
from jax import lax
from jax.experimental import pallas as pl
from jax.experimental.pallas import tpu as pltpu
from jax.experimental.pallas import tpu_sc as plsc

_INFO = plsc.get_sparse_core_info()
_NC, _NS = _INFO.num_cores, _INFO.num_subcores
_NW = _NC * _NS

_B = 16384 * 50
_D = 64
_BPW = _B // _NW

_SUB = 128
_NSUB = 4
_CB = _SUB * _NSUB
_NBLK = _BPW // _CB


def _body(table_hbm, idx_hbm, out_hbm, idx_v, rows_v, idx_sem, gat_sem):
    wid = lax.axis_index("s") * _NC + lax.axis_index("c")
    base = wid * _BPW

    @pl.loop(0, _NBLK)
    def _blk(b):
        off = base + b * _CB
        pltpu.sync_copy(idx_hbm.at[pl.ds(off, _CB)], idx_v)
        waits = []
        for j in range(_NSUB):
            waits.append(
                pltpu.async_copy(
                    table_hbm.at[idx_v.at[j]],
                    rows_v.at[pl.ds(j * _SUB, _SUB)],
                    gat_sem,
                )
            )
        for w in waits:
            w.wait()
        pltpu.sync_copy(rows_v, out_hbm.at[pl.ds(off, _CB)])


@jax.jit
def _run(iid_flat, table):
    mesh = plsc.VectorSubcoreMesh(core_axis_name="c", subcore_axis_name="s")
    k = pl.kernel(
        _body,
        out_type=jax.ShapeDtypeStruct((_B, _D), jnp.float32),
        mesh=mesh,
        scratch_types=[
            pltpu.VMEM((_NSUB, _SUB), jnp.int32),
            pltpu.VMEM((_CB, _D), jnp.float32),
            pltpu.SemaphoreType.DMA,
            pltpu.SemaphoreType.DMA,
        ],
    )
    return k(table, iid_flat)


def kernel(iid, table):
    out = _run(iid.reshape(_B).astype(jnp.int32), table)
    return out.reshape(iid.shape + (_D,))


# speedup vs baseline: 1.7974x; 1.7974x over previous
import functools

import jax
import jax.numpy as jnp
from jax import lax
from jax.experimental import pallas as pl
from jax.experimental.pallas import tpu as pltpu
from jax.experimental.pallas import tpu_sc as plsc

_INFO = plsc.get_sparse_core_info()
_NC, _NS = _INFO.num_cores, _INFO.num_subcores
_NW = _NC * _NS

_B = 16384 * 50
_D = 64
_BPW = _B // _NW

_SUB = 128
_NSUB = 4
_CB = _SUB * _NSUB
_NBLK = _BPW // _CB


def _body(table_hbm, idx_hbm, out_hbm, idx_v, rows_v, idx_sem, gat_sem):
    wid = lax.axis_index("s") * _NC + lax.axis_index("c")
    base = wid * _BPW

    @pl.loop(0, _NBLK)
    def _blk(b):
        off = base + b * _CB
        pltpu.sync_copy(idx_hbm.at[wid * _NBLK + b], idx_v)
        waits = []
        for j in range(_NSUB):
            waits.append(
                pltpu.async_copy(
                    table_hbm.at[idx_v.at[j]],
                    rows_v.at[pl.ds(j * _SUB, _SUB)],
                    gat_sem,
                )
            )
        for w in waits:
            w.wait()
        pltpu.sync_copy(rows_v, out_hbm.at[pl.ds(off, _CB)])


@jax.jit
def _run(iid_flat, table):
    mesh = plsc.VectorSubcoreMesh(core_axis_name="c", subcore_axis_name="s")
    k = pl.kernel(
        _body,
        out_type=jax.ShapeDtypeStruct((_B, _D), jnp.float32),
        mesh=mesh,
        scratch_types=[
            pltpu.VMEM((_NSUB, _SUB), jnp.int32),
            pltpu.VMEM((_CB, _D), jnp.float32),
            pltpu.SemaphoreType.DMA,
            pltpu.SemaphoreType.DMA,
        ],
        compiler_params=pltpu.CompilerParams(use_tc_tiling_on_sc=False),
    )
    return k(table, iid_flat)


def kernel(iid, table):
    idx3 = iid.reshape(_B // _CB, _NSUB, _SUB).astype(jnp.int32)
    out = _run(idx3, table)
    return out.reshape(iid.shape + (_D,))


# speedup vs baseline: 1.8748x; 1.0431x over previous
import jax
import jax.numpy as jnp
from jax import lax
from jax.experimental import pallas as pl
from jax.experimental.pallas import tpu as pltpu
from jax.experimental.pallas import tpu_sc as plsc

_INFO = plsc.get_sparse_core_info()
_NC, _NS = _INFO.num_cores, _INFO.num_subcores
_NW = _NC * _NS

_B = 16384 * 50
_D = 64
_BPW = _B // _NW

_SUB = 128
_NSUB = 4
_CB = _SUB * _NSUB
_NBLK = _BPW // _CB
_NBUF = 3


def _body(table_hbm, idx_hbm, out_hbm, idx_v, rows_v,
          g0, g1, g2, o0, o1, o2, isem):
    wid = lax.axis_index("s") * _NC + lax.axis_index("c")
    base_blk = wid * _NBLK
    gsems = (g0, g1, g2)
    osems = (o0, o1, o2)

    def prefetch_idx(b, bi):
        g = base_blk + jnp.minimum(b, _NBLK - 1)
        pltpu.async_copy(idx_hbm.at[g], idx_v.at[bi], isem)

    def wait_idx(bi):
        pltpu.make_async_copy(idx_hbm.at[base_blk], idx_v.at[bi], isem).wait()

    def fire(bi):
        for j in range(_NSUB):
            pltpu.async_copy(
                table_hbm.at[idx_v.at[bi].at[j]],
                rows_v.at[bi].at[pl.ds(j * _SUB, _SUB)],
                gsems[bi],
            )

    def wait_gathers(bi):
        for j in range(_NSUB):
            pltpu.make_async_copy(
                table_hbm.at[idx_v.at[bi].at[j]],
                rows_v.at[bi].at[pl.ds(j * _SUB, _SUB)],
                gsems[bi],
            ).wait()

    def store_async(b, bi):
        off = (base_blk + b) * _CB
        pltpu.async_copy(rows_v.at[bi], out_hbm.at[pl.ds(off, _CB)], osems[bi])

    def wait_store(bi):
        pltpu.make_async_copy(rows_v.at[bi], out_hbm.at[pl.ds(0, _CB)],
                              osems[bi]).wait()

    def substep(b, bi, do_ws):
        ib1 = (bi + 1) % _NBUF
        ib2 = (bi + 2) % _NBUF
        wait_idx(ib1)
        if do_ws:
            wait_store(ib1)
        fire(ib1)
        wait_gathers(bi)
        prefetch_idx(b + 2, ib2)
        store_async(b, bi)

    pltpu.sync_copy(idx_hbm.at[base_blk], idx_v.at[0])
    fire(0)
    prefetch_idx(1, 1)
    substep(0, 0, False)
    substep(1, 1, False)

    @pl.loop(0, (_NBLK - 3 - 2) // _NBUF)
    def _p(p):
        for s in range(_NBUF):
            substep(2 + p * _NBUF + s, (2 + s) % _NBUF, True)

    substep(47, 47 % _NBUF, True)
    substep(48, 48 % _NBUF, True)
    last_bi = (_NBLK - 1) % _NBUF
    wait_gathers(last_bi)
    store_async(_NBLK - 1, last_bi)
    wait_idx(_NBLK % _NBUF)
    for b in (_NBLK - 3, _NBLK - 2, _NBLK - 1):
        wait_store(b % _NBUF)


@jax.jit
def _run(idx3, table):
    mesh = plsc.VectorSubcoreMesh(core_axis_name="c", subcore_axis_name="s")
    k = pl.kernel(
        _body,
        out_type=jax.ShapeDtypeStruct((_B, _D), jnp.float32),
        mesh=mesh,
        scratch_types=[
            pltpu.VMEM((_NBUF, _NSUB, _SUB), jnp.int32),
            pltpu.VMEM((_NBUF, _CB, _D), jnp.float32),
        ] + [pltpu.SemaphoreType.DMA] * 7,
        compiler_params=pltpu.CompilerParams(use_tc_tiling_on_sc=False),
    )
    return k(table, idx3)


def kernel(iid, table):
    idx3 = iid.reshape(_B // _CB, _NSUB, _SUB).astype(jnp.int32)
    out = _run(idx3, table)
    return out.reshape(iid.shape + (_D,))
